# baseline (speedup 1.0000x reference)
import functools

import jax
import jax.numpy as jnp
from jax import lax
from jax.experimental import pallas as pl
from jax.experimental.pallas import tpu as pltpu
from jax.experimental.pallas import tpu_sc as plsc

B = 16384
D = 32
NC = 2
NS = 16
NW = NC * NS
BPW = B // NW
GROUP = 32

_sc_mesh = plsc.VectorSubcoreMesh(core_axis_name="c", subcore_axis_name="s")


@functools.partial(
    pl.kernel,
    mesh=_sc_mesh,
    out_type=[
        jax.ShapeDtypeStruct((B, D), jnp.float32),
        jax.ShapeDtypeStruct((B, D), jnp.float32),
    ],
    scratch_types=[
        pltpu.VMEM((BPW,), jnp.int32),
        pltpu.VMEM((BPW,), jnp.int32),
        pltpu.SemaphoreType.DMA,
    ],
)
def _sc_gather(uid_hbm, iid_hbm, utab_hbm, itab_hbm, uout_hbm, iout_hbm,
               uidx_v, iidx_v, sem):
    wid = lax.axis_index("s") * NC + lax.axis_index("c")
    base = wid * BPW
    pltpu.sync_copy(uid_hbm.at[pl.ds(base, BPW)], uidx_v)
    pltpu.sync_copy(iid_hbm.at[pl.ds(base, BPW)], iidx_v)

    def step(k, _):
        copies = []
        for half in range(2):
            row0 = (k * 2 + half) * GROUP
            for sub in range(GROUP // 16):
                off = row0 + sub * 16
                uvec = uidx_v[pl.ds(off, 16)]
                ivec = iidx_v[pl.ds(off, 16)]
                for l in range(16):
                    dst = base + off + l
                    copies.append(pltpu.async_copy(
                        utab_hbm.at[pl.ds(uvec[l], 1)],
                        uout_hbm.at[pl.ds(dst, 1)], sem))
                    copies.append(pltpu.async_copy(
                        itab_hbm.at[pl.ds(ivec[l], 1)],
                        iout_hbm.at[pl.ds(dst, 1)], sem))
        for cp in copies:
            cp.wait()
        return _

    lax.fori_loop(0, BPW // (2 * GROUP), step, None)


BLK = 1024


def _mlp_body(xu_ref, xv_ref, w1a_ref, w1b_ref, b1_ref, w2_ref, b2_ref,
              w3_ref, b3_ref, out_ref):
    h = jnp.dot(xu_ref[...], w1a_ref[...], preferred_element_type=jnp.float32)
    h = h + jnp.dot(xv_ref[...], w1b_ref[...], preferred_element_type=jnp.float32)
    h = jnp.maximum(h + b1_ref[...], 0.0)
    h2 = jnp.dot(h, w2_ref[...], preferred_element_type=jnp.float32)
    h2 = jnp.maximum(h2 + b2_ref[...], 0.0)
    out_ref[...] = jnp.sum(h2 * w3_ref[...], axis=1, keepdims=True) + b3_ref[...]


_mlp = pl.pallas_call(
    _mlp_body,
    grid=(B // BLK,),
    in_specs=[
        pl.BlockSpec((BLK, D), lambda i: (i, 0)),
        pl.BlockSpec((BLK, D), lambda i: (i, 0)),
        pl.BlockSpec((D, 64), lambda i: (0, 0)),
        pl.BlockSpec((D, 64), lambda i: (0, 0)),
        pl.BlockSpec((1, 64), lambda i: (0, 0)),
        pl.BlockSpec((64, 32), lambda i: (0, 0)),
        pl.BlockSpec((1, 32), lambda i: (0, 0)),
        pl.BlockSpec((1, 32), lambda i: (0, 0)),
        pl.BlockSpec((1, 1), lambda i: (0, 0)),
    ],
    out_specs=pl.BlockSpec((BLK, 1), lambda i: (i, 0)),
    out_shape=jax.ShapeDtypeStruct((B, 1), jnp.float32),
)


def kernel(user_ids, item_ids, user_table, item_table, W1, b1, W2, b2, W3, b3):
    uid = user_ids.astype(jnp.int32)
    iid = item_ids.astype(jnp.int32)
    urows, irows = _sc_gather(uid, iid, user_table, item_table)
    out = _mlp(urows, irows, W1[:D], W1[D:], b1.reshape(1, 64), W2,
               b2.reshape(1, 32), W3.reshape(1, 32), b3.reshape(1, 1))
    return out[:, 0]


# speedup vs baseline: 1.1982x; 1.1982x over previous
import functools

import jax
import jax.numpy as jnp
from jax import lax
from jax.experimental import pallas as pl
from jax.experimental.pallas import tpu as pltpu
from jax.experimental.pallas import tpu_sc as plsc

B = 16384
D = 32
NROWS = 1000000
PACK = 4
NC = 2
NS = 16
NW = NC * NS
BPW = B // NW
CHUNK = 128
NCH = BPW // CHUNK

_sc_mesh = plsc.VectorSubcoreMesh(core_axis_name="c", subcore_axis_name="s")


@functools.partial(
    pl.kernel,
    mesh=_sc_mesh,
    out_type=[
        jax.ShapeDtypeStruct((B, D), jnp.float32),
        jax.ShapeDtypeStruct((B, D), jnp.float32),
    ],
    scratch_types=[
        pltpu.VMEM((NCH, CHUNK), jnp.int32),
        pltpu.VMEM((NCH, CHUNK), jnp.int32),
        pltpu.VMEM((NCH, CHUNK), jnp.int32),
        pltpu.VMEM((NCH, CHUNK), jnp.int32),
        pltpu.VMEM((CHUNK, 128), jnp.float32),
        pltpu.VMEM((CHUNK, 128), jnp.float32),
        pltpu.VMEM((CHUNK, D), jnp.float32),
        pltpu.VMEM((CHUNK, D), jnp.float32),
        pltpu.SemaphoreType.DMA,
        pltpu.SemaphoreType.DMA,
    ],
)
def _sc_gather(uid_hbm, iid_hbm, utab_hbm, itab_hbm, uout_hbm, iout_hbm,
               uidx_v, iidx_v, utix_v, itix_v, uraw_v, iraw_v,
               uout_v, iout_v, usem, isem):
    wid = lax.axis_index("s") * NC + lax.axis_index("c")
    idx_row = wid * NCH
    base = wid * BPW
    pltpu.sync_copy(uid_hbm.at[pl.ds(idx_row, NCH)], uidx_v)
    pltpu.sync_copy(iid_hbm.at[pl.ds(idx_row, NCH)], iidx_v)
    for ch in range(NCH):
        for g in range(CHUNK // 16):
            sl = pl.ds(g * 16, 16)
            utix_v[ch, sl] = lax.shift_right_logical(uidx_v[ch, sl], 2)
            itix_v[ch, sl] = lax.shift_right_logical(iidx_v[ch, sl], 2)

    def chunk_body(ch, _):
        cu = pltpu.async_copy(utab_hbm.at[utix_v.at[ch]], uraw_v, usem)
        ci = pltpu.async_copy(itab_hbm.at[itix_v.at[ch]], iraw_v, isem)
        cu.wait()
        ci.wait()
        for g in range(CHUNK // 16):
            uoff = (uidx_v[ch, pl.ds(g * 16, 16)] & 3) * D
            ioff = (iidx_v[ch, pl.ds(g * 16, 16)] & 3) * D
            for l in range(16):
                j = g * 16 + l
                uo = uoff[l]
                io = ioff[l]
                uout_v[j, pl.ds(0, 16)] = uraw_v[j, pl.ds(uo, 16)]
                uout_v[j, pl.ds(16, 16)] = uraw_v[j, pl.ds(uo + 16, 16)]
                iout_v[j, pl.ds(0, 16)] = iraw_v[j, pl.ds(io, 16)]
                iout_v[j, pl.ds(16, 16)] = iraw_v[j, pl.ds(io + 16, 16)]
        pltpu.sync_copy(uout_v, uout_hbm.at[pl.ds(base + ch * CHUNK, CHUNK)])
        pltpu.sync_copy(iout_v, iout_hbm.at[pl.ds(base + ch * CHUNK, CHUNK)])
        return _

    lax.fori_loop(0, NCH, chunk_body, None)


BLK = 1024


def _mlp_body(xu_ref, xv_ref, w1a_ref, w1b_ref, b1_ref, w2_ref, b2_ref,
              w3_ref, b3_ref, out_ref):
    h = jnp.dot(xu_ref[...], w1a_ref[...], preferred_element_type=jnp.float32)
    h = h + jnp.dot(xv_ref[...], w1b_ref[...], preferred_element_type=jnp.float32)
    h = jnp.maximum(h + b1_ref[...], 0.0)
    h2 = jnp.dot(h, w2_ref[...], preferred_element_type=jnp.float32)
    h2 = jnp.maximum(h2 + b2_ref[...], 0.0)
    out_ref[...] = jnp.sum(h2 * w3_ref[...], axis=1, keepdims=True) + b3_ref[...]


_mlp = pl.pallas_call(
    _mlp_body,
    grid=(B // BLK,),
    in_specs=[
        pl.BlockSpec((BLK, D), lambda i: (i, 0)),
        pl.BlockSpec((BLK, D), lambda i: (i, 0)),
        pl.BlockSpec((D, 64), lambda i: (0, 0)),
        pl.BlockSpec((D, 64), lambda i: (0, 0)),
        pl.BlockSpec((1, 64), lambda i: (0, 0)),
        pl.BlockSpec((64, 32), lambda i: (0, 0)),
        pl.BlockSpec((1, 32), lambda i: (0, 0)),
        pl.BlockSpec((1, 32), lambda i: (0, 0)),
        pl.BlockSpec((1, 1), lambda i: (0, 0)),
    ],
    out_specs=pl.BlockSpec((BLK, 1), lambda i: (i, 0)),
    out_shape=jax.ShapeDtypeStruct((B, 1), jnp.float32),
)


def kernel(user_ids, item_ids, user_table, item_table, W1, b1, W2, b2, W3, b3):
    uid = user_ids.astype(jnp.int32).reshape(B // CHUNK, CHUNK)
    iid = item_ids.astype(jnp.int32).reshape(B // CHUNK, CHUNK)
    cu = user_table.reshape(NROWS // PACK, PACK * D)
    ci = item_table.reshape(NROWS // PACK, PACK * D)
    urows, irows = _sc_gather(uid, iid, cu, ci)
    out = _mlp(urows, irows, W1[:D], W1[D:], b1.reshape(1, 64), W2,
               b2.reshape(1, 32), W3.reshape(1, 32), b3.reshape(1, 1))
    return out[:, 0]
